# baseline (speedup 1.0000x reference)
import jax
import jax.numpy as jnp
from jax.experimental import pallas as pl
from jax.experimental.pallas import tpu as pltpu
from functools import partial

DIM = 384
QK = 384
HEADS = 8
CH = DIM // HEADS
NWIN = 7
P2 = NWIN * NWIN
WS = 8
W2 = WS * WS
TOPK = 4
SCALE = QK ** (-0.5)
N = 4
NWTOT = N * P2
TOK = NWTOT * W2
ROWBLK = 256


def _qkv_kernel(x_ref, w_ref, b_ref, q_ref, k_ref, v_ref):
    acc = jnp.dot(x_ref[...], w_ref[...], preferred_element_type=jnp.float32)
    acc = acc + b_ref[...]
    q_ref[...] = acc[:, :QK] * SCALE
    k_ref[...] = acc[:, QK:2 * QK]
    v_ref[...] = acc[:, 2 * QK:]


def _route_kernel(q_ref, k_ref, idx_ref):
    n = pl.program_id(0)
    qw = jnp.mean(q_ref[0], axis=1)
    kw = jnp.mean(k_ref[0], axis=1)
    logits = jnp.dot(qw, kw.T, preferred_element_type=jnp.float32)
    col = jax.lax.broadcasted_iota(jnp.int32, (P2, P2), 1)
    cols = []
    for _ in range(TOPK):
        am = jnp.argmax(logits, axis=-1).astype(jnp.int32)
        cols.append(am[:, None])
        logits = jnp.where(col == am[:, None], -jnp.inf, logits)
    idx = jnp.concatenate(cols, axis=1)
    idx_ref[0] = idx + n * P2


def _attn_kernel(idx_ref, q_ref, k_ref, v_ref, o_ref, logit_s, v_s):
    t = pl.program_id(1)
    q = q_ref[0]
    k = k_ref[0]
    v_s[t] = v_ref[0]
    for h in range(HEADS):
        qh = q[:, h * CH:(h + 1) * CH]
        kh = k[:, h * CH:(h + 1) * CH]
        logit_s[t, h] = jnp.dot(qh, kh.T, preferred_element_type=jnp.float32)

    @pl.when(t == TOPK - 1)
    def _():
        for h in range(HEADS):
            l = jnp.concatenate([logit_s[tt, h] for tt in range(TOPK)],
                                axis=-1)
            m = jnp.max(l, axis=-1, keepdims=True)
            p = jnp.exp(l - m)
            p = p / jnp.sum(p, axis=-1, keepdims=True)
            vh = v_s[:, :, h * CH:(h + 1) * CH].reshape(TOPK * W2, CH)
            o_ref[0, :, h * CH:(h + 1) * CH] = jnp.dot(
                p, vh, preferred_element_type=jnp.float32)


def _lepe_kernel(v_ref, w_ref, b_ref, o_ref):
    v = v_ref[0]
    o_ref[0] = jnp.zeros((56, 56, DIM), jnp.float32) + b_ref[0]
    for dy in range(3):
        for dx in range(3):
            wv = w_ref[dy * 3 + dx]
            oy0, oy1 = max(0, 1 - dy), 56 - max(0, dy - 1)
            ox0, ox1 = max(0, 1 - dx), 56 - max(0, dx - 1)
            iy0, iy1 = oy0 + dy - 1, oy1 + dy - 1
            ix0, ix1 = ox0 + dx - 1, ox1 + dx - 1
            o_ref[0, oy0:oy1, ox0:ox1, :] += v[iy0:iy1, ix0:ix1, :] * wv


def _proj_kernel(a_ref, l_ref, w_ref, b_ref, o_ref):
    s = a_ref[...] + l_ref[...]
    o_ref[...] = jnp.dot(s, w_ref[...], preferred_element_type=jnp.float32) + b_ref[...]


def _win2img(a):
    C = a.shape[-1]
    return (a.reshape(N, NWIN, NWIN, WS, WS, C)
             .transpose(0, 1, 3, 2, 4, 5)
             .reshape(N, 56, 56, C))


def kernel(x, qkv_w, qkv_b, wo_w, wo_b, lepe_w, lepe_b):
    xw = (x.reshape(N, NWIN, WS, NWIN, WS, DIM)
           .transpose(0, 1, 3, 2, 4, 5)
           .reshape(TOK, DIM))
    wqkvT = qkv_w.T
    woT = wo_w.T
    lw = lepe_w[:, 0].transpose(1, 2, 0).reshape(9, DIM)

    q, k, v = pl.pallas_call(
        _qkv_kernel,
        grid=(TOK // ROWBLK,),
        in_specs=[
            pl.BlockSpec((ROWBLK, DIM), lambda i: (i, 0)),
            pl.BlockSpec((DIM, 3 * DIM), lambda i: (0, 0)),
            pl.BlockSpec((1, 3 * DIM), lambda i: (0, 0)),
        ],
        out_specs=[
            pl.BlockSpec((ROWBLK, QK), lambda i: (i, 0)),
            pl.BlockSpec((ROWBLK, QK), lambda i: (i, 0)),
            pl.BlockSpec((ROWBLK, DIM), lambda i: (i, 0)),
        ],
        out_shape=[
            jax.ShapeDtypeStruct((TOK, QK), jnp.float32),
            jax.ShapeDtypeStruct((TOK, QK), jnp.float32),
            jax.ShapeDtypeStruct((TOK, DIM), jnp.float32),
        ],
    )(xw, wqkvT, qkv_b[None, :])

    qw = q.reshape(N, P2, W2, QK)
    kw = k.reshape(N, P2, W2, QK)

    r_idx = pl.pallas_call(
        _route_kernel,
        grid=(N,),
        in_specs=[
            pl.BlockSpec((1, P2, W2, QK), lambda n: (n, 0, 0, 0)),
            pl.BlockSpec((1, P2, W2, QK), lambda n: (n, 0, 0, 0)),
        ],
        out_specs=pl.BlockSpec((1, P2, TOPK), lambda n: (n, 0, 0)),
        out_shape=jax.ShapeDtypeStruct((N, P2, TOPK), jnp.int32),
    )(qw, kw)
    idx_flat = r_idx.reshape(NWTOT * TOPK)

    q3 = q.reshape(NWTOT, W2, QK)
    k3 = k.reshape(NWTOT, W2, QK)
    v3 = v.reshape(NWTOT, W2, DIM)
    attn_out = pl.pallas_call(
        _attn_kernel,
        grid_spec=pltpu.PrefetchScalarGridSpec(
            num_scalar_prefetch=1,
            grid=(NWTOT, TOPK),
            in_specs=[
                pl.BlockSpec((1, W2, QK), lambda w, t, idx: (w, 0, 0)),
                pl.BlockSpec((1, W2, QK), lambda w, t, idx: (idx[w * TOPK + t], 0, 0)),
                pl.BlockSpec((1, W2, DIM), lambda w, t, idx: (idx[w * TOPK + t], 0, 0)),
            ],
            out_specs=pl.BlockSpec((1, W2, DIM), lambda w, t, idx: (w, 0, 0)),
            scratch_shapes=[
                pltpu.VMEM((TOPK, HEADS, W2, W2), jnp.float32),
                pltpu.VMEM((TOPK, W2, DIM), jnp.float32),
            ],
        ),
        out_shape=jax.ShapeDtypeStruct((NWTOT, W2, DIM), jnp.float32),
    )(idx_flat, q3, k3, v3)

    v_img = _win2img(v.reshape(N, P2, W2, DIM))
    lepe = pl.pallas_call(
        _lepe_kernel,
        grid=(N,),
        in_specs=[
            pl.BlockSpec((1, 56, 56, DIM), lambda n: (n, 0, 0, 0)),
            pl.BlockSpec((9, DIM), lambda n: (0, 0)),
            pl.BlockSpec((1, DIM), lambda n: (0, 0)),
        ],
        out_specs=pl.BlockSpec((1, 56, 56, DIM), lambda n: (n, 0, 0, 0)),
        out_shape=jax.ShapeDtypeStruct((N, 56, 56, DIM), jnp.float32),
    )(v_img, lw, lepe_b[None, :])

    a_img = _win2img(attn_out.reshape(N, P2, W2, DIM)).reshape(TOK, DIM)
    out = pl.pallas_call(
        _proj_kernel,
        grid=(TOK // ROWBLK,),
        in_specs=[
            pl.BlockSpec((ROWBLK, DIM), lambda i: (i, 0)),
            pl.BlockSpec((ROWBLK, DIM), lambda i: (i, 0)),
            pl.BlockSpec((DIM, DIM), lambda i: (0, 0)),
            pl.BlockSpec((1, DIM), lambda i: (0, 0)),
        ],
        out_specs=pl.BlockSpec((ROWBLK, DIM), lambda i: (i, 0)),
        out_shape=jax.ShapeDtypeStruct((TOK, DIM), jnp.float32),
    )(a_img, lepe.reshape(TOK, DIM), woT, wo_b[None, :])

    return out.reshape(N, 56, 56, DIM)


# speedup vs baseline: 1.8394x; 1.8394x over previous
import jax
import jax.numpy as jnp
from jax.experimental import pallas as pl
from jax.experimental.pallas import tpu as pltpu

DIM = 384
QK = 384
HEADS = 8
CH = DIM // HEADS
NWIN = 7
P2 = NWIN * NWIN
WS = 8
W2 = WS * WS
TOPK = 4
SCALE = QK ** (-0.5)
N = 4


def _qkv_kernel(x_ref, w_ref, b_ref, q_ref, k_ref, v_ref, vi_ref):
    xb = x_ref[0].reshape(WS * 56, DIM)
    acc = jnp.dot(xb, w_ref[...], preferred_element_type=jnp.float32)
    acc = acc + b_ref[...]
    vi_ref[0] = acc[:, 2 * QK:].reshape(WS, 56, DIM)
    acc3 = acc.reshape(WS, 56, 3 * DIM)
    for i in range(NWIN):
        blk = acc3[:, i * WS:(i + 1) * WS, :].reshape(W2, 3 * DIM)
        q_ref[0, i] = blk[:, :QK] * SCALE
        k_ref[0, i] = blk[:, QK:2 * QK]
        v_ref[0, i] = blk[:, 2 * QK:]


def _route_kernel(q_ref, k_ref, idx_ref):
    qw = jnp.mean(q_ref[0], axis=1)
    kw = jnp.mean(k_ref[0], axis=1)
    logits = jax.lax.dot_general(qw, kw, (((1,), (1,)), ((), ())),
                                 preferred_element_type=jnp.float32)
    col = jax.lax.broadcasted_iota(jnp.int32, (P2, P2), 1)
    cols = []
    for _ in range(TOPK):
        am = jnp.argmax(logits, axis=-1).astype(jnp.int32)
        cols.append(am[:, None])
        logits = jnp.where(col == am[:, None], -jnp.inf, logits)
    idx_ref[0, 0] = jnp.concatenate(cols, axis=1)


def _attn_kernel(idx_ref, q_ref, k_ref, v_ref, o_ref):
    n = pl.program_id(0)
    base = n * P2 * TOPK

    def body(w, _):
        q = q_ref[0, w]
        iv = [idx_ref[base + w * TOPK + t] for t in range(TOPK)]
        kcat = jnp.concatenate([k_ref[0, i] for i in iv], axis=0)
        vcat = jnp.concatenate([v_ref[0, i] for i in iv], axis=0)
        parts = []
        for h in range(HEADS):
            hs = slice(h * CH, (h + 1) * CH)
            l = jax.lax.dot_general(q[:, hs], kcat[:, hs],
                                    (((1,), (1,)), ((), ())),
                                    preferred_element_type=jnp.float32)
            m = jnp.max(l, axis=-1, keepdims=True)
            p = jnp.exp(l - m)
            s = jnp.sum(p, axis=-1, keepdims=True)
            oh = jnp.dot(p, vcat[:, hs], preferred_element_type=jnp.float32)
            parts.append(oh / s)
        ocat = jnp.concatenate(parts, axis=-1)
        j = w // NWIN
        i = w - j * NWIN
        o_ref[0, pl.ds(j * WS, WS), pl.ds(i * WS, WS), :] = (
            ocat.reshape(WS, WS, DIM))
        return 0

    jax.lax.fori_loop(0, P2, body, 0)


def _tail_kernel(a_ref, v_ref, lw_ref, lb_ref, w_ref, b_ref, o_ref, scr):
    v = v_ref[0]
    scr[...] = jnp.zeros((56, 56, DIM), jnp.float32) + lb_ref[0]
    for dy in range(3):
        for dx in range(3):
            wv = lw_ref[dy * 3 + dx]
            oy0, oy1 = max(0, 1 - dy), 56 - max(0, dy - 1)
            ox0, ox1 = max(0, 1 - dx), 56 - max(0, dx - 1)
            iy0, iy1 = oy0 + dy - 1, oy1 + dy - 1
            ix0, ix1 = ox0 + dx - 1, ox1 + dx - 1
            scr[oy0:oy1, ox0:ox1, :] += v[iy0:iy1, ix0:ix1, :] * wv
    s = (a_ref[0] + scr[...]).reshape(56 * 56, DIM)
    out = jnp.dot(s, w_ref[...], preferred_element_type=jnp.float32) + b_ref[...]
    o_ref[0] = out.reshape(56, 56, DIM)


def kernel(x, qkv_w, qkv_b, wo_w, wo_b, lepe_w, lepe_b):
    wqkvT = qkv_w.T
    woT = wo_w.T
    lw = lepe_w[:, 0].transpose(1, 2, 0).reshape(9, DIM)

    q, k, v, v_img = pl.pallas_call(
        _qkv_kernel,
        grid=(N, NWIN),
        in_specs=[
            pl.BlockSpec((1, WS, 56, DIM), lambda n, j: (n, j, 0, 0)),
            pl.BlockSpec((DIM, 3 * DIM), lambda n, j: (0, 0)),
            pl.BlockSpec((1, 3 * DIM), lambda n, j: (0, 0)),
        ],
        out_specs=[
            pl.BlockSpec((1, NWIN, W2, QK), lambda n, j: (n, j, 0, 0)),
            pl.BlockSpec((1, NWIN, W2, QK), lambda n, j: (n, j, 0, 0)),
            pl.BlockSpec((1, NWIN, W2, DIM), lambda n, j: (n, j, 0, 0)),
            pl.BlockSpec((1, WS, 56, DIM), lambda n, j: (n, j, 0, 0)),
        ],
        out_shape=[
            jax.ShapeDtypeStruct((N, P2, W2, QK), jnp.float32),
            jax.ShapeDtypeStruct((N, P2, W2, QK), jnp.float32),
            jax.ShapeDtypeStruct((N, P2, W2, DIM), jnp.float32),
            jax.ShapeDtypeStruct((N, 56, 56, DIM), jnp.float32),
        ],
    )(x, wqkvT, qkv_b[None, :])

    r_idx = pl.pallas_call(
        _route_kernel,
        grid=(N,),
        in_specs=[
            pl.BlockSpec((1, P2, W2, QK), lambda n: (n, 0, 0, 0)),
            pl.BlockSpec((1, P2, W2, QK), lambda n: (n, 0, 0, 0)),
        ],
        out_specs=pl.BlockSpec((1, 1, P2, TOPK), lambda n: (n, 0, 0, 0)),
        out_shape=jax.ShapeDtypeStruct((N, 1, P2, TOPK), jnp.int32),
    )(q, k)
    idx_flat = r_idx.reshape(N * P2 * TOPK)

    attn_img = pl.pallas_call(
        _attn_kernel,
        grid_spec=pltpu.PrefetchScalarGridSpec(
            num_scalar_prefetch=1,
            grid=(N,),
            in_specs=[
                pl.BlockSpec((1, P2, W2, QK), lambda n, idx: (n, 0, 0, 0)),
                pl.BlockSpec((1, P2, W2, QK), lambda n, idx: (n, 0, 0, 0)),
                pl.BlockSpec((1, P2, W2, DIM), lambda n, idx: (n, 0, 0, 0)),
            ],
            out_specs=pl.BlockSpec((1, 56, 56, DIM), lambda n, idx: (n, 0, 0, 0)),
        ),
        out_shape=jax.ShapeDtypeStruct((N, 56, 56, DIM), jnp.float32),
    )(idx_flat, q, k, v)

    out = pl.pallas_call(
        _tail_kernel,
        grid=(N,),
        in_specs=[
            pl.BlockSpec((1, 56, 56, DIM), lambda n: (n, 0, 0, 0)),
            pl.BlockSpec((1, 56, 56, DIM), lambda n: (n, 0, 0, 0)),
            pl.BlockSpec((9, DIM), lambda n: (0, 0)),
            pl.BlockSpec((1, DIM), lambda n: (0, 0)),
            pl.BlockSpec((DIM, DIM), lambda n: (0, 0)),
            pl.BlockSpec((1, DIM), lambda n: (0, 0)),
        ],
        out_specs=pl.BlockSpec((1, 56, 56, DIM), lambda n: (n, 0, 0, 0)),
        out_shape=jax.ShapeDtypeStruct((N, 56, 56, DIM), jnp.float32),
        scratch_shapes=[pltpu.VMEM((56, 56, DIM), jnp.float32)],
    )(attn_img, v_img, lw, lepe_b[None, :], woT, wo_b[None, :])

    return out


# speedup vs baseline: 4.0557x; 2.2049x over previous
import jax
import jax.numpy as jnp
from jax.experimental import pallas as pl
from jax.experimental.pallas import tpu as pltpu

DIM = 384
QK = 384
HEADS = 8
CH = DIM // HEADS
HP = 128
QP = HEADS * HP
NWIN = 7
P2 = NWIN * NWIN
WS = 8
W2 = WS * WS
TOPK = 4
SCALE = QK ** (-0.5)
N = 4
ACOLS = 3 * QP + DIM


def _qkv_kernel(x_ref, w_ref, b_ref, q_ref, k_ref, v_ref, vi_ref, qs_ref, ks_ref):
    xb = x_ref[0].reshape(WS * 56, DIM)
    acc = jnp.dot(xb, w_ref[...], preferred_element_type=jnp.float32)
    acc = acc + b_ref[...]
    vi_ref[0] = acc[:, 3 * QP:].reshape(WS, 56, DIM)
    acc3 = acc.reshape(WS, 56, ACOLS)
    for i in range(NWIN):
        blk = acc3[:, i * WS:(i + 1) * WS, :].reshape(W2, ACOLS)
        q_ref[0, i] = blk[:, :QP]
        k_ref[0, i] = blk[:, QP:2 * QP]
        v_ref[0, i] = blk[:, 2 * QP:3 * QP]
        qs_ref[0, i, 0] = jnp.sum(blk[:, :QP], axis=0)
        ks_ref[0, i, 0] = jnp.sum(blk[:, QP:2 * QP], axis=0)


def _route_kernel(qs_ref, ks_ref, idx_ref):
    qs = qs_ref[0][:, 0, :]
    ks = ks_ref[0][:, 0, :]
    logits = jax.lax.dot_general(qs, ks, (((1,), (1,)), ((), ())),
                                 preferred_element_type=jnp.float32)
    col = jax.lax.broadcasted_iota(jnp.int32, (P2, P2), 1)
    cols = []
    for _ in range(TOPK):
        am = jnp.argmax(logits, axis=-1).astype(jnp.int32)
        cols.append(am[:, None])
        logits = jnp.where(col == am[:, None], -jnp.inf, logits)
    idx_ref[0, 0] = jnp.concatenate(cols, axis=1)


def _attn_kernel(idx_ref, q_ref, k_ref, v_ref, o_ref):
    n = pl.program_id(0)
    j = pl.program_id(1)
    base = (n * P2 + j * NWIN) * TOPK

    def stage_qk(i):
        q = q_ref[0, i]
        iv = [idx_ref[base + i * TOPK + t] for t in range(TOPK)]
        kcat = jnp.concatenate([k_ref[0, t] for t in iv], axis=0)
        vcat = jnp.concatenate([v_ref[0, t] for t in iv], axis=0)
        ls = [jax.lax.dot_general(q[:, h * HP:(h + 1) * HP],
                                  kcat[:, h * HP:(h + 1) * HP],
                                  (((1,), (1,)), ((), ())),
                                  preferred_element_type=jnp.float32)
              for h in range(HEADS)]
        return ls, vcat

    def stage_m(ls):
        mm = ls[0]
        for l in ls[1:]:
            mm = jnp.maximum(mm, l)
        return jnp.max(mm, axis=-1, keepdims=True)

    def stage_out(i, ls, m, vcat):
        parts = []
        for h in range(HEADS):
            p = jnp.exp(ls[h] - m)
            oa = jnp.dot(p, vcat[:, h * HP:(h + 1) * HP],
                         preferred_element_type=jnp.float32)
            parts.append(oa[:, :CH] / oa[:, CH:CH + 1])
        ocat = jnp.concatenate(parts, axis=-1)
        o_ref[0, :, i * WS:(i + 1) * WS, :] = ocat.reshape(WS, WS, DIM)

    for ia, ib in ((0, 1), (2, 3), (4, 5)):
        lsa, vca = stage_qk(ia)
        lsb, vcb = stage_qk(ib)
        ma = stage_m(lsa)
        mb = stage_m(lsb)
        stage_out(ia, lsa, ma, vca)
        stage_out(ib, lsb, mb, vcb)
    ls, vc = stage_qk(6)
    stage_out(6, ls, stage_m(ls), vc)


def _tail_kernel(a_ref, v_ref, lw_ref, lb_ref, w_ref, b_ref, o_ref, scr):
    v = v_ref[0]
    scr[...] = jnp.zeros((56, 56, DIM), jnp.float32) + lb_ref[0]
    for dy in range(3):
        for dx in range(3):
            wv = lw_ref[dy * 3 + dx]
            oy0, oy1 = max(0, 1 - dy), 56 - max(0, dy - 1)
            ox0, ox1 = max(0, 1 - dx), 56 - max(0, dx - 1)
            iy0, iy1 = oy0 + dy - 1, oy1 + dy - 1
            ix0, ix1 = ox0 + dx - 1, ox1 + dx - 1
            scr[oy0:oy1, ox0:ox1, :] += v[iy0:iy1, ix0:ix1, :] * wv
    s = (a_ref[0] + scr[...]).reshape(56 * 56, DIM)
    out = jnp.dot(s, w_ref[...], preferred_element_type=jnp.float32) + b_ref[...]
    o_ref[0] = out.reshape(56, 56, DIM)


def _pad_heads(w):
    w3 = w.reshape(w.shape[:-1] + (HEADS, CH))
    pad = [(0, 0)] * (w3.ndim - 1) + [(0, HP - CH)]
    return jnp.pad(w3, pad).reshape(w.shape[:-1] + (QP,))


def kernel(x, qkv_w, qkv_b, wo_w, wo_b, lepe_w, lepe_b):
    wqkvT = qkv_w.T
    wbig = jnp.concatenate([
        _pad_heads(wqkvT[:, :QK] * SCALE),
        _pad_heads(wqkvT[:, QK:2 * QK]),
        _pad_heads(wqkvT[:, 2 * QK:]),
        wqkvT[:, 2 * QK:],
    ], axis=1)
    ones_col = jnp.zeros((QP,), jnp.float32).at[
        jnp.arange(HEADS) * HP + CH].set(1.0)
    bbig = jnp.concatenate([
        _pad_heads(qkv_b[:QK] * SCALE),
        _pad_heads(qkv_b[QK:2 * QK]),
        _pad_heads(qkv_b[2 * QK:]) + ones_col,
        qkv_b[2 * QK:],
    ])[None, :]
    woT = wo_w.T
    lw = lepe_w[:, 0].transpose(1, 2, 0).reshape(9, DIM)

    q, k, v, v_img, qs, ks = pl.pallas_call(
        _qkv_kernel,
        grid=(N, NWIN),
        in_specs=[
            pl.BlockSpec((1, WS, 56, DIM), lambda n, j: (n, j, 0, 0)),
            pl.BlockSpec((DIM, ACOLS), lambda n, j: (0, 0)),
            pl.BlockSpec((1, ACOLS), lambda n, j: (0, 0)),
        ],
        out_specs=[
            pl.BlockSpec((1, NWIN, W2, QP), lambda n, j: (n, j, 0, 0)),
            pl.BlockSpec((1, NWIN, W2, QP), lambda n, j: (n, j, 0, 0)),
            pl.BlockSpec((1, NWIN, W2, QP), lambda n, j: (n, j, 0, 0)),
            pl.BlockSpec((1, WS, 56, DIM), lambda n, j: (n, j, 0, 0)),
            pl.BlockSpec((1, NWIN, 1, QP), lambda n, j: (n, j, 0, 0)),
            pl.BlockSpec((1, NWIN, 1, QP), lambda n, j: (n, j, 0, 0)),
        ],
        out_shape=[
            jax.ShapeDtypeStruct((N, P2, W2, QP), jnp.float32),
            jax.ShapeDtypeStruct((N, P2, W2, QP), jnp.float32),
            jax.ShapeDtypeStruct((N, P2, W2, QP), jnp.float32),
            jax.ShapeDtypeStruct((N, 56, 56, DIM), jnp.float32),
            jax.ShapeDtypeStruct((N, P2, 1, QP), jnp.float32),
            jax.ShapeDtypeStruct((N, P2, 1, QP), jnp.float32),
        ],
    )(x, wbig, bbig)

    r_idx = pl.pallas_call(
        _route_kernel,
        grid=(N,),
        in_specs=[
            pl.BlockSpec((1, P2, 1, QP), lambda n: (n, 0, 0, 0)),
            pl.BlockSpec((1, P2, 1, QP), lambda n: (n, 0, 0, 0)),
        ],
        out_specs=pl.BlockSpec((1, 1, P2, TOPK), lambda n: (n, 0, 0, 0)),
        out_shape=jax.ShapeDtypeStruct((N, 1, P2, TOPK), jnp.int32),
    )(qs, ks)
    idx_flat = r_idx.reshape(N * P2 * TOPK)

    attn_img = pl.pallas_call(
        _attn_kernel,
        grid_spec=pltpu.PrefetchScalarGridSpec(
            num_scalar_prefetch=1,
            grid=(N, NWIN),
            in_specs=[
                pl.BlockSpec((1, NWIN, W2, QP), lambda n, j, idx: (n, j, 0, 0)),
                pl.BlockSpec((1, P2, W2, QP), lambda n, j, idx: (n, 0, 0, 0)),
                pl.BlockSpec((1, P2, W2, QP), lambda n, j, idx: (n, 0, 0, 0)),
            ],
            out_specs=pl.BlockSpec((1, WS, 56, DIM),
                                   lambda n, j, idx: (n, j, 0, 0)),
        ),
        out_shape=jax.ShapeDtypeStruct((N, 56, 56, DIM), jnp.float32),
    )(idx_flat, q, k, v)

    out = pl.pallas_call(
        _tail_kernel,
        grid=(N,),
        in_specs=[
            pl.BlockSpec((1, 56, 56, DIM), lambda n: (n, 0, 0, 0)),
            pl.BlockSpec((1, 56, 56, DIM), lambda n: (n, 0, 0, 0)),
            pl.BlockSpec((9, DIM), lambda n: (0, 0)),
            pl.BlockSpec((1, DIM), lambda n: (0, 0)),
            pl.BlockSpec((DIM, DIM), lambda n: (0, 0)),
            pl.BlockSpec((1, DIM), lambda n: (0, 0)),
        ],
        out_specs=pl.BlockSpec((1, 56, 56, DIM), lambda n: (n, 0, 0, 0)),
        out_shape=jax.ShapeDtypeStruct((N, 56, 56, DIM), jnp.float32),
        scratch_shapes=[pltpu.VMEM((56, 56, DIM), jnp.float32)],
    )(attn_img, v_img, lw, lepe_b[None, :], woT, wo_b[None, :])

    return out


# speedup vs baseline: 4.4901x; 1.1071x over previous
import jax
import jax.numpy as jnp
from jax.experimental import pallas as pl
from jax.experimental.pallas import tpu as pltpu

DIM = 384
QK = 384
HEADS = 8
CH = DIM // HEADS
HP = 128
QP = HEADS * HP
NWIN = 7
P2 = NWIN * NWIN
WS = 8
W2 = WS * WS
TOPK = 4
SCALE = QK ** (-0.5)
N = 4
ACOLS = 3 * QP + DIM


def _qkv_kernel(x_ref, w_ref, b_ref, q_ref, k_ref, v_ref, vi_ref, qs_ref, ks_ref):
    xb = x_ref[0].reshape(WS * 56, DIM)
    acc = jnp.dot(xb, w_ref[...], preferred_element_type=jnp.float32)
    acc = acc + b_ref[...]
    vi_ref[0] = acc[:, 3 * QP:].reshape(WS, 56, DIM)
    acc3 = acc.reshape(WS, 56, ACOLS)
    for i in range(NWIN):
        blk = acc3[:, i * WS:(i + 1) * WS, :].reshape(W2, ACOLS)
        q_ref[0, i] = blk[:, :QP].astype(jnp.bfloat16)
        k_ref[0, i] = blk[:, QP:2 * QP].astype(jnp.bfloat16)
        v_ref[0, i] = blk[:, 2 * QP:3 * QP].astype(jnp.bfloat16)
        qs_ref[0, i, 0] = jnp.sum(blk[:, :QP], axis=0)
        ks_ref[0, i, 0] = jnp.sum(blk[:, QP:2 * QP], axis=0)


def _route_kernel(qs_ref, ks_ref, idx_ref):
    qs = qs_ref[0][:, 0, :]
    ks = ks_ref[0][:, 0, :]
    logits = jax.lax.dot_general(qs, ks, (((1,), (1,)), ((), ())),
                                 preferred_element_type=jnp.float32)
    col = jax.lax.broadcasted_iota(jnp.int32, (P2, P2), 1)
    cols = []
    for _ in range(TOPK):
        am = jnp.argmax(logits, axis=-1).astype(jnp.int32)
        cols.append(am[:, None])
        logits = jnp.where(col == am[:, None], -jnp.inf, logits)
    idx_ref[0, 0] = jnp.concatenate(cols, axis=1)


def _attn_kernel(idx_ref, q_ref, k_ref, v_ref, o_ref):
    n = pl.program_id(0)
    j = pl.program_id(1)
    base = (n * P2 + j * NWIN) * TOPK

    def stage_qk(i):
        q = q_ref[0, i]
        iv = [idx_ref[base + i * TOPK + t] for t in range(TOPK)]
        kcat = jnp.concatenate([k_ref[0, t] for t in iv], axis=0)
        vcat = jnp.concatenate([v_ref[0, t] for t in iv], axis=0)
        ls = [jax.lax.dot_general(q[:, h * HP:(h + 1) * HP],
                                  kcat[:, h * HP:(h + 1) * HP],
                                  (((1,), (1,)), ((), ())),
                                  preferred_element_type=jnp.float32)
              for h in range(HEADS)]
        return ls, vcat

    def stage_m(ls):
        mm = ls[0]
        for l in ls[1:]:
            mm = jnp.maximum(mm, l)
        return jnp.max(mm, axis=-1, keepdims=True)

    def stage_out(i, ls, m, vcat):
        parts = []
        for h in range(HEADS):
            p = jnp.exp(ls[h] - m).astype(jnp.bfloat16)
            oa = jnp.dot(p, vcat[:, h * HP:(h + 1) * HP],
                         preferred_element_type=jnp.float32)
            parts.append(oa[:, :CH] / oa[:, CH:CH + 1])
        ocat = jnp.concatenate(parts, axis=-1)
        o_ref[0, :, i * WS:(i + 1) * WS, :] = ocat.reshape(WS, WS, DIM)

    for ia, ib in ((0, 1), (2, 3), (4, 5)):
        lsa, vca = stage_qk(ia)
        lsb, vcb = stage_qk(ib)
        ma = stage_m(lsa)
        mb = stage_m(lsb)
        stage_out(ia, lsa, ma, vca)
        stage_out(ib, lsb, mb, vcb)
    ls, vc = stage_qk(6)
    stage_out(6, ls, stage_m(ls), vc)


def _tail_kernel(a_ref, v_ref, lw_ref, lb_ref, w_ref, b_ref, o_ref, scr):
    v = v_ref[0]
    scr[...] = jnp.zeros((56, 56, DIM), jnp.float32) + lb_ref[0]
    for dy in range(3):
        for dx in range(3):
            wv = lw_ref[dy * 3 + dx]
            oy0, oy1 = max(0, 1 - dy), 56 - max(0, dy - 1)
            ox0, ox1 = max(0, 1 - dx), 56 - max(0, dx - 1)
            iy0, iy1 = oy0 + dy - 1, oy1 + dy - 1
            ix0, ix1 = ox0 + dx - 1, ox1 + dx - 1
            scr[oy0:oy1, ox0:ox1, :] += v[iy0:iy1, ix0:ix1, :] * wv
    s = (a_ref[0] + scr[...]).reshape(56 * 56, DIM)
    out = jnp.dot(s, w_ref[...], preferred_element_type=jnp.float32) + b_ref[...]
    o_ref[0] = out.reshape(56, 56, DIM)


def _pad_heads(w):
    w3 = w.reshape(w.shape[:-1] + (HEADS, CH))
    pad = [(0, 0)] * (w3.ndim - 1) + [(0, HP - CH)]
    return jnp.pad(w3, pad).reshape(w.shape[:-1] + (QP,))


def kernel(x, qkv_w, qkv_b, wo_w, wo_b, lepe_w, lepe_b):
    wqkvT = qkv_w.T
    wbig = jnp.concatenate([
        _pad_heads(wqkvT[:, :QK] * SCALE),
        _pad_heads(wqkvT[:, QK:2 * QK]),
        _pad_heads(wqkvT[:, 2 * QK:]),
        wqkvT[:, 2 * QK:],
    ], axis=1)
    ones_col = jnp.zeros((QP,), jnp.float32).at[
        jnp.arange(HEADS) * HP + CH].set(1.0)
    bbig = jnp.concatenate([
        _pad_heads(qkv_b[:QK] * SCALE),
        _pad_heads(qkv_b[QK:2 * QK]),
        _pad_heads(qkv_b[2 * QK:]) + ones_col,
        qkv_b[2 * QK:],
    ])[None, :]
    woT = wo_w.T
    lw = lepe_w[:, 0].transpose(1, 2, 0).reshape(9, DIM)

    q, k, v, v_img, qs, ks = pl.pallas_call(
        _qkv_kernel,
        grid=(N, NWIN),
        in_specs=[
            pl.BlockSpec((1, WS, 56, DIM), lambda n, j: (n, j, 0, 0)),
            pl.BlockSpec((DIM, ACOLS), lambda n, j: (0, 0)),
            pl.BlockSpec((1, ACOLS), lambda n, j: (0, 0)),
        ],
        out_specs=[
            pl.BlockSpec((1, NWIN, W2, QP), lambda n, j: (n, j, 0, 0)),
            pl.BlockSpec((1, NWIN, W2, QP), lambda n, j: (n, j, 0, 0)),
            pl.BlockSpec((1, NWIN, W2, QP), lambda n, j: (n, j, 0, 0)),
            pl.BlockSpec((1, WS, 56, DIM), lambda n, j: (n, j, 0, 0)),
            pl.BlockSpec((1, NWIN, 1, QP), lambda n, j: (n, j, 0, 0)),
            pl.BlockSpec((1, NWIN, 1, QP), lambda n, j: (n, j, 0, 0)),
        ],
        out_shape=[
            jax.ShapeDtypeStruct((N, P2, W2, QP), jnp.bfloat16),
            jax.ShapeDtypeStruct((N, P2, W2, QP), jnp.bfloat16),
            jax.ShapeDtypeStruct((N, P2, W2, QP), jnp.bfloat16),
            jax.ShapeDtypeStruct((N, 56, 56, DIM), jnp.float32),
            jax.ShapeDtypeStruct((N, P2, 1, QP), jnp.float32),
            jax.ShapeDtypeStruct((N, P2, 1, QP), jnp.float32),
        ],
    )(x, wbig, bbig)

    r_idx = pl.pallas_call(
        _route_kernel,
        grid=(N,),
        in_specs=[
            pl.BlockSpec((1, P2, 1, QP), lambda n: (n, 0, 0, 0)),
            pl.BlockSpec((1, P2, 1, QP), lambda n: (n, 0, 0, 0)),
        ],
        out_specs=pl.BlockSpec((1, 1, P2, TOPK), lambda n: (n, 0, 0, 0)),
        out_shape=jax.ShapeDtypeStruct((N, 1, P2, TOPK), jnp.int32),
    )(qs, ks)
    idx_flat = r_idx.reshape(N * P2 * TOPK)

    attn_img = pl.pallas_call(
        _attn_kernel,
        grid_spec=pltpu.PrefetchScalarGridSpec(
            num_scalar_prefetch=1,
            grid=(N, NWIN),
            in_specs=[
                pl.BlockSpec((1, NWIN, W2, QP), lambda n, j, idx: (n, j, 0, 0)),
                pl.BlockSpec((1, P2, W2, QP), lambda n, j, idx: (n, 0, 0, 0)),
                pl.BlockSpec((1, P2, W2, QP), lambda n, j, idx: (n, 0, 0, 0)),
            ],
            out_specs=pl.BlockSpec((1, WS, 56, DIM),
                                   lambda n, j, idx: (n, j, 0, 0)),
        ),
        out_shape=jax.ShapeDtypeStruct((N, 56, 56, DIM), jnp.float32),
    )(idx_flat, q, k, v)

    out = pl.pallas_call(
        _tail_kernel,
        grid=(N,),
        in_specs=[
            pl.BlockSpec((1, 56, 56, DIM), lambda n: (n, 0, 0, 0)),
            pl.BlockSpec((1, 56, 56, DIM), lambda n: (n, 0, 0, 0)),
            pl.BlockSpec((9, DIM), lambda n: (0, 0)),
            pl.BlockSpec((1, DIM), lambda n: (0, 0)),
            pl.BlockSpec((DIM, DIM), lambda n: (0, 0)),
            pl.BlockSpec((1, DIM), lambda n: (0, 0)),
        ],
        out_specs=pl.BlockSpec((1, 56, 56, DIM), lambda n: (n, 0, 0, 0)),
        out_shape=jax.ShapeDtypeStruct((N, 56, 56, DIM), jnp.float32),
        scratch_shapes=[pltpu.VMEM((56, 56, DIM), jnp.float32)],
    )(attn_img, v_img, lw, lepe_b[None, :], woT, wo_b[None, :])

    return out


# speedup vs baseline: 4.5334x; 1.0097x over previous
import jax
import jax.numpy as jnp
from jax.experimental import pallas as pl
from jax.experimental.pallas import tpu as pltpu

DIM = 384
QK = 384
HEADS = 8
CH = DIM // HEADS
HP = 128
QP = HEADS * HP
NWIN = 7
P2 = NWIN * NWIN
WS = 8
W2 = WS * WS
TOPK = 4
SCALE = QK ** (-0.5)
N = 4
ACOLS = 3 * QP + DIM


def _qkv_kernel(x_ref, w_ref, b_ref, q_ref, k_ref, v_ref, vi_ref, qs_ref, ks_ref):
    xb = x_ref[0].reshape(WS * 56, DIM)
    acc = jnp.dot(xb, w_ref[...], preferred_element_type=jnp.float32)
    acc = acc + b_ref[...]
    vi_ref[0] = acc[:, 3 * QP:].reshape(WS, 56, DIM)
    acc3 = acc.reshape(WS, 56, ACOLS)
    for i in range(NWIN):
        blk = acc3[:, i * WS:(i + 1) * WS, :].reshape(W2, ACOLS)
        q_ref[0, i] = blk[:, :QP].astype(jnp.bfloat16)
        k_ref[0, i] = blk[:, QP:2 * QP].astype(jnp.bfloat16)
        v_ref[0, i] = blk[:, 2 * QP:3 * QP].astype(jnp.bfloat16)
        qs_ref[0, i, 0] = jnp.sum(blk[:, :QP], axis=0)
        ks_ref[0, i, 0] = jnp.sum(blk[:, QP:2 * QP], axis=0)


def _route_kernel(qs_ref, ks_ref, idx_ref):
    qs = qs_ref[0][:, 0, :]
    ks = ks_ref[0][:, 0, :]
    logits = jax.lax.dot_general(qs, ks, (((1,), (1,)), ((), ())),
                                 preferred_element_type=jnp.float32)
    col = jax.lax.broadcasted_iota(jnp.int32, (P2, P2), 1)
    cols = []
    for _ in range(TOPK):
        am = jnp.argmax(logits, axis=-1).astype(jnp.int32)
        cols.append(am[:, None])
        logits = jnp.where(col == am[:, None], -jnp.inf, logits)
    idx_ref[0, 0] = jnp.concatenate(cols, axis=1)


def _attn_kernel(idx_ref, q_ref, k_ref, v_ref, o_ref):
    n = pl.program_id(0)
    j = pl.program_id(1)
    base = (n * P2 + j * NWIN) * TOPK

    def stage_qk(i):
        q = q_ref[0, i]
        iv = [idx_ref[base + i * TOPK + t] for t in range(TOPK)]
        kcat = jnp.concatenate([k_ref[0, t] for t in iv], axis=0)
        vcat = jnp.concatenate([v_ref[0, t] for t in iv], axis=0)
        ls = [jax.lax.dot_general(q[:, h * HP:(h + 1) * HP].astype(jnp.float32),
                                  kcat[:, h * HP:(h + 1) * HP].astype(jnp.float32),
                                  (((1,), (1,)), ((), ())),
                                  preferred_element_type=jnp.float32)
              for h in range(HEADS)]
        return ls, vcat

    def stage_m(ls):
        mm = ls[0]
        for l in ls[1:]:
            mm = jnp.maximum(mm, l)
        return jnp.max(mm, axis=-1, keepdims=True)

    def stage_out(i, ls, m, vcat):
        parts = []
        for h in range(HEADS):
            p = jnp.exp(ls[h] - m)
            oa = jnp.dot(p, vcat[:, h * HP:(h + 1) * HP].astype(jnp.float32),
                         preferred_element_type=jnp.float32)
            parts.append(oa[:, :CH] / oa[:, CH:CH + 1])
        ocat = jnp.concatenate(parts, axis=-1)
        o_ref[0, :, i * WS:(i + 1) * WS, :] = ocat.reshape(WS, WS, DIM)

    for ia, ib in ((0, 1), (2, 3), (4, 5)):
        lsa, vca = stage_qk(ia)
        lsb, vcb = stage_qk(ib)
        ma = stage_m(lsa)
        mb = stage_m(lsb)
        stage_out(ia, lsa, ma, vca)
        stage_out(ib, lsb, mb, vcb)
    ls, vc = stage_qk(6)
    stage_out(6, ls, stage_m(ls), vc)


def _tail_kernel(a_ref, v_ref, lw_ref, lb_ref, w_ref, b_ref, o_ref, scr):
    v = v_ref[0]
    scr[...] = jnp.zeros((56, 56, DIM), jnp.float32) + lb_ref[0]
    for dy in range(3):
        for dx in range(3):
            wv = lw_ref[dy * 3 + dx]
            oy0, oy1 = max(0, 1 - dy), 56 - max(0, dy - 1)
            ox0, ox1 = max(0, 1 - dx), 56 - max(0, dx - 1)
            iy0, iy1 = oy0 + dy - 1, oy1 + dy - 1
            ix0, ix1 = ox0 + dx - 1, ox1 + dx - 1
            scr[oy0:oy1, ox0:ox1, :] += v[iy0:iy1, ix0:ix1, :] * wv
    s = (a_ref[0] + scr[...]).reshape(56 * 56, DIM)
    out = jnp.dot(s, w_ref[...], preferred_element_type=jnp.float32) + b_ref[...]
    o_ref[0] = out.reshape(56, 56, DIM)


def _pad_heads(w):
    w3 = w.reshape(w.shape[:-1] + (HEADS, CH))
    pad = [(0, 0)] * (w3.ndim - 1) + [(0, HP - CH)]
    return jnp.pad(w3, pad).reshape(w.shape[:-1] + (QP,))


def kernel(x, qkv_w, qkv_b, wo_w, wo_b, lepe_w, lepe_b):
    wqkvT = qkv_w.T
    wbig = jnp.concatenate([
        _pad_heads(wqkvT[:, :QK] * SCALE),
        _pad_heads(wqkvT[:, QK:2 * QK]),
        _pad_heads(wqkvT[:, 2 * QK:]),
        wqkvT[:, 2 * QK:],
    ], axis=1)
    ones_col = jnp.zeros((QP,), jnp.float32).at[
        jnp.arange(HEADS) * HP + CH].set(1.0)
    bbig = jnp.concatenate([
        _pad_heads(qkv_b[:QK] * SCALE),
        _pad_heads(qkv_b[QK:2 * QK]),
        _pad_heads(qkv_b[2 * QK:]) + ones_col,
        qkv_b[2 * QK:],
    ])[None, :]
    woT = wo_w.T
    lw = lepe_w[:, 0].transpose(1, 2, 0).reshape(9, DIM)

    q, k, v, v_img, qs, ks = pl.pallas_call(
        _qkv_kernel,
        grid=(N, NWIN),
        in_specs=[
            pl.BlockSpec((1, WS, 56, DIM), lambda n, j: (n, j, 0, 0)),
            pl.BlockSpec((DIM, ACOLS), lambda n, j: (0, 0)),
            pl.BlockSpec((1, ACOLS), lambda n, j: (0, 0)),
        ],
        out_specs=[
            pl.BlockSpec((1, NWIN, W2, QP), lambda n, j: (n, j, 0, 0)),
            pl.BlockSpec((1, NWIN, W2, QP), lambda n, j: (n, j, 0, 0)),
            pl.BlockSpec((1, NWIN, W2, QP), lambda n, j: (n, j, 0, 0)),
            pl.BlockSpec((1, WS, 56, DIM), lambda n, j: (n, j, 0, 0)),
            pl.BlockSpec((1, NWIN, 1, QP), lambda n, j: (n, j, 0, 0)),
            pl.BlockSpec((1, NWIN, 1, QP), lambda n, j: (n, j, 0, 0)),
        ],
        out_shape=[
            jax.ShapeDtypeStruct((N, P2, W2, QP), jnp.bfloat16),
            jax.ShapeDtypeStruct((N, P2, W2, QP), jnp.bfloat16),
            jax.ShapeDtypeStruct((N, P2, W2, QP), jnp.bfloat16),
            jax.ShapeDtypeStruct((N, 56, 56, DIM), jnp.float32),
            jax.ShapeDtypeStruct((N, P2, 1, QP), jnp.float32),
            jax.ShapeDtypeStruct((N, P2, 1, QP), jnp.float32),
        ],
    )(x, wbig, bbig)

    r_idx = pl.pallas_call(
        _route_kernel,
        grid=(N,),
        in_specs=[
            pl.BlockSpec((1, P2, 1, QP), lambda n: (n, 0, 0, 0)),
            pl.BlockSpec((1, P2, 1, QP), lambda n: (n, 0, 0, 0)),
        ],
        out_specs=pl.BlockSpec((1, 1, P2, TOPK), lambda n: (n, 0, 0, 0)),
        out_shape=jax.ShapeDtypeStruct((N, 1, P2, TOPK), jnp.int32),
    )(qs, ks)
    idx_flat = r_idx.reshape(N * P2 * TOPK)

    attn_img = pl.pallas_call(
        _attn_kernel,
        grid_spec=pltpu.PrefetchScalarGridSpec(
            num_scalar_prefetch=1,
            grid=(N, NWIN),
            in_specs=[
                pl.BlockSpec((1, NWIN, W2, QP), lambda n, j, idx: (n, j, 0, 0)),
                pl.BlockSpec((1, P2, W2, QP), lambda n, j, idx: (n, 0, 0, 0)),
                pl.BlockSpec((1, P2, W2, QP), lambda n, j, idx: (n, 0, 0, 0)),
            ],
            out_specs=pl.BlockSpec((1, WS, 56, DIM),
                                   lambda n, j, idx: (n, j, 0, 0)),
        ),
        out_shape=jax.ShapeDtypeStruct((N, 56, 56, DIM), jnp.float32),
    )(idx_flat, q, k, v)

    out = pl.pallas_call(
        _tail_kernel,
        grid=(N,),
        in_specs=[
            pl.BlockSpec((1, 56, 56, DIM), lambda n: (n, 0, 0, 0)),
            pl.BlockSpec((1, 56, 56, DIM), lambda n: (n, 0, 0, 0)),
            pl.BlockSpec((9, DIM), lambda n: (0, 0)),
            pl.BlockSpec((1, DIM), lambda n: (0, 0)),
            pl.BlockSpec((DIM, DIM), lambda n: (0, 0)),
            pl.BlockSpec((1, DIM), lambda n: (0, 0)),
        ],
        out_specs=pl.BlockSpec((1, 56, 56, DIM), lambda n: (n, 0, 0, 0)),
        out_shape=jax.ShapeDtypeStruct((N, 56, 56, DIM), jnp.float32),
        scratch_shapes=[pltpu.VMEM((56, 56, DIM), jnp.float32)],
    )(attn_img, v_img, lw, lepe_b[None, :], woT, wo_b[None, :])

    return out
